# baseline (speedup 1.0000x reference)
import functools

import jax
import jax.numpy as jnp
from jax import lax
from jax.experimental import pallas as pl
from jax.experimental.pallas import tpu as pltpu
from jax.experimental.pallas import tpu_sc as plsc

_BB = 8


def _idx_body(z_ref, e_ref, i_ref):
    e = e_ref[...]
    K, C = e.shape
    HW = z_ref.shape[2]
    e_sq = jnp.sum(e * e, axis=1)[:, None]
    em2b = (e * (-2.0)).astype(jnp.bfloat16)
    iota_f = lax.broadcasted_iota(jnp.int32, (K, HW), 0).astype(jnp.float32)
    for b in range(_BB):
        zc = z_ref[b]
        s2 = lax.dot_general(em2b, zc.astype(jnp.bfloat16),
                             (((1,), (0,)), ((), ())),
                             preferred_element_type=jnp.float32)
        d = s2 + e_sq
        m = jnp.min(d, axis=0, keepdims=True)
        ival = jnp.where(d == m, iota_f, float(K))
        i_ref[b] = jnp.min(ival, axis=0).astype(jnp.int32)


def _tc_indices(z, embedding):
    B, C, HW = z.shape
    K = embedding.shape[0]
    return pl.pallas_call(
        _idx_body,
        grid=(B // _BB,),
        in_specs=[
            pl.BlockSpec((_BB, C, HW), lambda b: (b, 0, 0)),
            pl.BlockSpec((K, C), lambda b: (0, 0)),
        ],
        out_specs=pl.BlockSpec((_BB, HW), lambda b: (b, 0)),
        out_shape=jax.ShapeDtypeStruct((B, HW), jnp.int32),
    )(z, embedding)


def _sc_gather(embedding, idx):
    K, C = embedding.shape
    B, HW = idx.shape
    L = 16
    info = plsc.get_sparse_core_info()
    nc, ns = info.num_cores, info.num_subcores
    mesh = plsc.VectorSubcoreMesh(core_axis_name="c", subcore_axis_name="s")

    nw = nc * ns
    wpb = nw // B
    cs = C // wpb

    @functools.partial(
        pl.kernel, mesh=mesh,
        compiler_params=pltpu.CompilerParams(needs_layout_passes=False),
        out_type=jax.ShapeDtypeStruct((B, C, HW), jnp.float32),
        scratch_types=[
            pltpu.VMEM((HW,), jnp.int32),
            pltpu.VMEM((K * C,), jnp.float32),
            pltpu.VMEM((cs, HW), jnp.float32),
        ],
    )
    def k(e_hbm, idx_hbm, out_hbm, idx_v, e_v, out_v):
        wid = lax.axis_index("s") * nc + lax.axis_index("c")
        b = wid // wpb
        c0 = (wid % wpb) * cs
        pltpu.sync_copy(idx_hbm.at[b], idx_v)
        pltpu.sync_copy(e_hbm, e_v)

        @plsc.parallel_loop(0, HW // L, unroll=2)
        def body(j):
            row = idx_v[pl.ds(j * L, L)]
            for c in range(cs):
                out_v[c, pl.ds(j * L, L)] = plsc.load_gather(
                    e_v, [row + (c0 + c) * K])
        pltpu.sync_copy(out_v, out_hbm.at[b, pl.ds(c0, cs), :])

    return k(embedding.T.reshape(C * K), idx)


def kernel(z_e, embedding):
    B, C, H, W = z_e.shape
    z = z_e.reshape(B, C, H * W)
    half = B // 2
    idx0 = _tc_indices(z[:half], embedding)
    idx1 = _tc_indices(z[half:], embedding)
    out0 = _sc_gather(embedding, idx0)
    out1 = _sc_gather(embedding, idx1)
    out = jnp.concatenate([out0, out1], axis=0)
    return out.reshape(B, C, H, W)


# speedup vs baseline: 1.2855x; 1.2855x over previous
import functools

import jax
import jax.numpy as jnp
from jax import lax
from jax.experimental import pallas as pl
from jax.experimental.pallas import tpu as pltpu
from jax.experimental.pallas import tpu_sc as plsc

_BB = 8


def _idx_body(z_ref, e_ref, i_ref):
    e = e_ref[...]
    K, C = e.shape
    HW = z_ref.shape[2]
    e_sq = jnp.sum(e * e, axis=1)[:, None]
    em2b = (e * (-2.0)).astype(jnp.bfloat16)
    iota_f = lax.broadcasted_iota(jnp.int32, (K, HW), 0).astype(jnp.float32)
    for b in range(_BB):
        zc = z_ref[b]
        s2 = lax.dot_general(em2b, zc.astype(jnp.bfloat16),
                             (((1,), (0,)), ((), ())),
                             preferred_element_type=jnp.float32)
        d = s2 + e_sq
        m = jnp.min(d, axis=0, keepdims=True)
        ival = jnp.where(d == m, iota_f, float(K))
        i_ref[b] = jnp.min(ival, axis=0).astype(jnp.int32)


def _tc_indices(z, embedding):
    B, C, HW = z.shape
    K = embedding.shape[0]
    return pl.pallas_call(
        _idx_body,
        grid=(B // _BB,),
        in_specs=[
            pl.BlockSpec((_BB, C, HW), lambda b: (b, 0, 0)),
            pl.BlockSpec((K, C), lambda b: (0, 0)),
        ],
        out_specs=pl.BlockSpec((_BB, HW), lambda b: (b, 0)),
        out_shape=jax.ShapeDtypeStruct((B, HW), jnp.int32),
    )(z, embedding)


def _sc_gather(embedding, idx):
    K, C = embedding.shape
    B, HW = idx.shape
    L = 16
    info = plsc.get_sparse_core_info()
    nc, ns = info.num_cores, info.num_subcores
    mesh = plsc.VectorSubcoreMesh(core_axis_name="c", subcore_axis_name="s")

    nw = nc * ns
    wpb = nw // B
    cs = C // wpb

    @functools.partial(
        pl.kernel, mesh=mesh,
        compiler_params=pltpu.CompilerParams(needs_layout_passes=False),
        out_type=jax.ShapeDtypeStruct((B, C, HW), jnp.float32),
        scratch_types=[
            pltpu.VMEM((HW,), jnp.int32),
            pltpu.VMEM((K * C,), jnp.float32),
            pltpu.VMEM((cs, HW), jnp.float32),
        ],
    )
    def k(e_hbm, idx_hbm, out_hbm, idx_v, e_v, out_v):
        wid = lax.axis_index("s") * nc + lax.axis_index("c")
        b = wid // wpb
        c0 = (wid % wpb) * cs
        pltpu.sync_copy(idx_hbm.at[b], idx_v)
        pltpu.sync_copy(e_hbm, e_v)

        @plsc.parallel_loop(0, HW // L, unroll=2)
        def body(j):
            row = idx_v[pl.ds(j * L, L)]
            for c in range(cs):
                out_v[c, pl.ds(j * L, L)] = plsc.load_gather(
                    e_v, [row + (c0 + c) * K])
        pltpu.sync_copy(out_v, out_hbm.at[b, pl.ds(c0, cs), :])

    return k(embedding.T.reshape(C * K), idx)


def kernel(z_e, embedding):
    B, C, H, W = z_e.shape
    z = z_e.reshape(B, C, H * W)
    idx = _tc_indices(z, embedding)
    out = _sc_gather(embedding, idx)
    return out.reshape(B, C, H, W)
